# Initial kernel scaffold; baseline (speedup 1.0000x reference)
import jax
import jax.numpy as jnp
from jax.experimental import pallas as pl


def kernel(x, preds, labels, centers):
    raise NotImplementedError("write your pallas kernel here")


# speedup vs baseline: 3.2738x; 3.2738x over previous
import functools

import jax
import jax.numpy as jnp
from jax.experimental import pallas as pl
from jax.experimental.pallas import tpu as pltpu

_EPS = 1e-6


def _loss_kernel(x_ref, preds_ref, labels_ref, centers_ref, out_ref, *, inv_batch):
    i = pl.program_id(0)
    x = x_ref[...]
    preds = preds_ref[...]
    labels = labels_ref[...]
    centers = centers_ref[...]

    b, c = preds.shape
    iota = jax.lax.broadcasted_iota(jnp.int32, (b, c), 1)
    onehot_l = iota == labels

    masked = jnp.where(onehot_l, -jnp.inf, preds)
    rowmax = jnp.max(masked, axis=1, keepdims=True)
    adv = jnp.min(jnp.where(masked == rowmax, iota, c), axis=1, keepdims=True)
    onehot_a = iota == adv

    y = x + _EPS
    yc = jax.lax.dot_general(y, centers, (((1,), (1,)), ((), ())),
                             preferred_element_type=jnp.float32)
    cn2 = jax.lax.dot_general(jnp.ones((1, y.shape[1]), jnp.float32),
                              centers * centers, (((1,), (1,)), ((), ())),
                              preferred_element_type=jnp.float32)
    yn2 = jnp.sum(y * y, axis=1, keepdims=True)

    t = cn2 - 2.0 * yc
    t_ap = jnp.sum(jnp.where(onehot_l, t, 0.0), axis=1, keepdims=True)
    t_an = jnp.sum(jnp.where(onehot_a, t, 0.0), axis=1, keepdims=True)
    d_ap = jnp.sqrt(jnp.maximum(yn2 + t_ap, 0.0))
    d_an = jnp.sqrt(jnp.maximum(yn2 + t_an, 0.0))
    part = jnp.sum(jnp.maximum(d_ap - d_an + 1.0, 0.0)) * inv_batch

    @pl.when(i == 0)
    def _():
        out_ref[0, 0] = 0.0

    out_ref[0, 0] += part


def kernel(x, preds, labels, centers):
    batch, feat = x.shape
    num_classes = centers.shape[0]
    blk = 256
    grid = batch // blk
    labels2 = labels.astype(jnp.int32).reshape(batch, 1)

    out = pl.pallas_call(
        functools.partial(_loss_kernel, inv_batch=1.0 / batch),
        grid=(grid,),
        in_specs=[
            pl.BlockSpec((blk, feat), lambda i: (i, 0)),
            pl.BlockSpec((blk, num_classes), lambda i: (i, 0)),
            pl.BlockSpec((blk, 1), lambda i: (i, 0)),
            pl.BlockSpec((num_classes, feat), lambda i: (0, 0)),
        ],
        out_specs=pl.BlockSpec(memory_space=pltpu.SMEM),
        out_shape=jax.ShapeDtypeStruct((1, 1), jnp.float32),
        compiler_params=pltpu.CompilerParams(
            dimension_semantics=("arbitrary",),
        ),
    )(x, preds, labels2, centers)
    return out[0, 0]
